# baseline (speedup 1.0000x reference)
import functools

import jax
import jax.numpy as jnp
from jax.experimental import pallas as pl

N = 50000
E = 800000
B = 256
L = 730
D_IN = 78
H = 128

_PREC = jax.lax.Precision.HIGHEST


def _bn_rows(x, g, b):
    m = jnp.mean(x, 0)
    v = jnp.mean((x - m) ** 2, 0)
    return g * (x - m) / jnp.sqrt(v + 1e-5) + b


def _head_body(zp_ref, xt_ref, w1_ref, b1_ref, g1_ref, bb1_ref,
               w2_ref, b2_ref, g2_ref, bb2_ref,
               w3_ref, b3_ref, g3_ref, bb3_ref,
               wo_ref, bo_ref, out_ref):
    xc = jnp.concatenate([zp_ref[...], xt_ref[...]], axis=1)
    h1 = jnp.dot(xc, w1_ref[...], precision=_PREC) + b1_ref[...]
    h1 = jax.nn.relu(_bn_rows(h1, g1_ref[...], bb1_ref[...]))
    h2 = jnp.dot(h1, w2_ref[...], precision=_PREC) + b2_ref[...]
    h2 = jax.nn.relu(_bn_rows(h2, g2_ref[...], bb2_ref[...]))
    h3 = jnp.dot(h2, w3_ref[...], precision=_PREC) + b3_ref[...]
    h3 = jax.nn.relu(_bn_rows(h3, g3_ref[...], bb3_ref[...]))
    out_ref[...] = jnp.dot(h3, wo_ref[...], precision=_PREC) + bo_ref[...]


def _head(zp, xt, p):
    return pl.pallas_call(
        _head_body,
        out_shape=jax.ShapeDtypeStruct((B, 1), jnp.float32),
    )(zp, xt, p['fc1_W'], p['fc1_b'], p['bnf1_g'], p['bnf1_b'],
      p['fc2_W'], p['fc2_b'], p['bnf2_g'], p['bnf2_b'],
      p['fc3_W'], p['fc3_b'], p['bnf3_g'], p['bnf3_b'],
      p['out_W'], p['out_b'])


def _bn_ncl(x, g, b):
    m = jnp.mean(x, (0, 2), keepdims=True)
    v = jnp.mean((x - m) ** 2, (0, 2), keepdims=True)
    return g[None, :, None] * (x - m) / jnp.sqrt(v + 1e-5) + b[None, :, None]


def _conv1d(x, W, b):
    y = jax.lax.conv_general_dilated(x, W, (1,), 'VALID',
                                     dimension_numbers=('NCH', 'OIH', 'NCH'))
    return y + b[None, :, None]


def _maxpool3(x):
    return jax.lax.reduce_window(x, -jnp.inf, jax.lax.max, (1, 1, 3), (1, 1, 3), 'VALID')


def kernel(x, edge_index, batch, target, params, eps):
    p = params
    loop = jnp.arange(N, dtype=edge_index.dtype)
    src = jnp.concatenate([edge_index[0], loop])
    dst = jnp.concatenate([edge_index[1], loop])
    deg = jax.ops.segment_sum(jnp.ones_like(dst, dtype=jnp.float32), dst, num_segments=N)
    dinv = jnp.where(deg > 0, 1.0 / jnp.sqrt(deg), 0.0)
    norm = dinv[src] * dinv[dst]

    def gcn(h, W, b):
        hw = h @ W
        msg = hw[src] * norm[:, None]
        return jax.ops.segment_sum(msg, dst, num_segments=N) + b

    identity = x @ p['res_W'] + p['res_b']
    h = jax.nn.relu(_bn_rows(gcn(x, p['conv1_W'], p['conv1_b']), p['bn1_g'], p['bn1_b']))
    h = jax.nn.relu(_bn_rows(gcn(h, p['conv2_W'], p['conv2_b']), p['bn2_g'], p['bn2_b']))
    h = jax.nn.relu(_bn_rows(gcn(h, p['conv3_W'], p['conv3_b']), p['bn3_g'], p['bn3_b']))
    h = jax.nn.relu(_bn_rows(gcn(h, p['conv4_W'], p['conv4_b']), p['bn4_g'], p['bn4_b']) + identity)
    mu = h @ p['mu_W'] + p['mu_b']
    logvar = h @ p['lv_W'] + p['lv_b']
    z = mu + eps * jnp.exp(0.5 * logvar)
    zp = jax.ops.segment_sum(z, batch, num_segments=B)
    m = jnp.mean(zp, -1, keepdims=True)
    v = jnp.mean((zp - m) ** 2, -1, keepdims=True)
    zp = p['ln_g'] * (zp - m) / jnp.sqrt(v + 1e-5) + p['ln_b']

    t = target[:, None, :]
    c = _maxpool3(jax.nn.relu(_bn_ncl(_conv1d(t, p['cxt1_W'], p['cxt1_b']), p['bnxt1_g'], p['bnxt1_b'])))
    c = _maxpool3(jax.nn.relu(_bn_ncl(_conv1d(c, p['cxt2_W'], p['cxt2_b']), p['bnxt2_g'], p['bnxt2_b'])))
    c = _maxpool3(jax.nn.relu(_bn_ncl(_conv1d(c, p['cxt3_W'], p['cxt3_b']), p['bnxt3_g'], p['bnxt3_b'])))
    xt = c.reshape(c.shape[0], -1) @ p['fc1xt_W'] + p['fc1xt_b']

    out = _head(zp, xt, p)
    return (out, zp)


# speedup vs baseline: 3.8208x; 3.8208x over previous
import functools

import jax
import jax.numpy as jnp
from jax import lax
from jax.experimental import pallas as pl
from jax.experimental.pallas import tpu as pltpu
from jax.experimental.pallas import tpu_sc as plsc

N = 50000
E = 800000
B = 256
L = 730
D_IN = 78
H = 128

_PREC = jax.lax.Precision.HIGHEST

QROWS = 8448
NQ = 6
NPAD = NQ * QROWS
UROWS = 50048
ZROW = 50000
TRASH = QROWS
BUFROWS = QROWS + 16
KB = 128
EPAD = 850048
NBLK = EPAD // KB
TROWS = QROWS // 16
ZROWS = 64
ZCHUNKS = [(o, min(ZROWS, TROWS - o)) for o in range(0, TROWS, ZROWS)]


def _propagate_body(u_hbm, srcs_hbm, dsts_hbm, tab_hbm, out_hbm,
                    tab_v, src_v, dst_v, idxl_v, rows_v, zero_v, buf_sh, sem):
    c = lax.axis_index("c")
    s = lax.axis_index("s")
    pltpu.sync_copy(tab_hbm, tab_v)
    tabs = [tab_v[pl.ds(16 * q, 16)] for q in range(NQ)]
    zvec = jnp.zeros((16,), jnp.float32)

    @pl.loop(0, ZROWS)
    def _(r):
        @pl.loop(0, H, step=16)
        def _(f):
            zero_v[r, pl.ds(f, 16)] = zvec

    for qi in range(NQ // 2):
        q = (NQ // 2) * c + qi
        qbase = q * QROWS
        tq = jnp.where(c == 0, tabs[qi], tabs[NQ // 2 + qi])
        sblk = tq[0]
        nblk = tq[1]

        for zo, zn in ZCHUNKS:
            zoff = pl.multiple_of(s * TROWS + zo, 8)
            pltpu.sync_copy(zero_v.at[pl.ds(0, zn)],
                            buf_sh.at[pl.ds(zoff, zn)])

        plsc.subcore_barrier()

        nmine = (nblk - s + 15) // 16

        @pl.loop(0, nmine)
        def _(i):
            blk = sblk + s + i * 16
            off = pl.multiple_of(blk * KB, KB)
            pltpu.sync_copy(srcs_hbm.at[pl.ds(off, KB)], src_v)
            pltpu.sync_copy(dsts_hbm.at[pl.ds(off, KB)], dst_v)
            for j in range(KB // 16):
                d = dst_v[pl.ds(j * 16, 16)]
                in_q = (d >= qbase) & (d < qbase + QROWS)
                loc = jnp.where(in_q, d - qbase, TRASH)
                idxl_v[pl.ds(j * 16, 16)] = loc
            pltpu.async_copy(u_hbm.at[src_v], rows_v, sem).wait()
            pltpu.sync_copy(rows_v, buf_sh.at[idxl_v], add=True)

        plsc.subcore_barrier()

        pltpu.sync_copy(buf_sh.at[pl.ds(pl.multiple_of(s * TROWS, 8), TROWS)],
                        out_hbm.at[pl.ds(pl.multiple_of(qbase + s * TROWS, 8), TROWS)])


def _propagate(u, srcs, dsts, tab):
    mesh = plsc.VectorSubcoreMesh(core_axis_name="c", subcore_axis_name="s")
    kern = pl.kernel(
        _propagate_body,
        out_type=jax.ShapeDtypeStruct((NPAD, H), jnp.float32),
        mesh=mesh,
        scratch_types=[
            pltpu.VMEM((16 * NQ,), jnp.int32),
            pltpu.VMEM((KB,), jnp.int32),
            pltpu.VMEM((KB,), jnp.int32),
            pltpu.VMEM((KB,), jnp.int32),
            pltpu.VMEM((KB, H), jnp.float32),
            pltpu.VMEM((ZROWS, H), jnp.float32),
            pltpu.VMEM_SHARED((BUFROWS, H), jnp.float32),
            pltpu.SemaphoreType.DMA,
        ],
    )
    return kern(u, srcs, dsts, tab)


def _bn_rows(x, g, b):
    m = jnp.mean(x, 0)
    v = jnp.mean((x - m) ** 2, 0)
    return g * (x - m) / jnp.sqrt(v + 1e-5) + b


def _head_body(zp_ref, xt_ref, w1_ref, b1_ref, g1_ref, bb1_ref,
               w2_ref, b2_ref, g2_ref, bb2_ref,
               w3_ref, b3_ref, g3_ref, bb3_ref,
               wo_ref, bo_ref, out_ref):
    xc = jnp.concatenate([zp_ref[...], xt_ref[...]], axis=1)
    h1 = jnp.dot(xc, w1_ref[...], precision=_PREC) + b1_ref[...]
    h1 = jax.nn.relu(_bn_rows(h1, g1_ref[...], bb1_ref[...]))
    h2 = jnp.dot(h1, w2_ref[...], precision=_PREC) + b2_ref[...]
    h2 = jax.nn.relu(_bn_rows(h2, g2_ref[...], bb2_ref[...]))
    h3 = jnp.dot(h2, w3_ref[...], precision=_PREC) + b3_ref[...]
    h3 = jax.nn.relu(_bn_rows(h3, g3_ref[...], bb3_ref[...]))
    out_ref[...] = jnp.dot(h3, wo_ref[...], precision=_PREC) + bo_ref[...]


def _head(zp, xt, p):
    return pl.pallas_call(
        _head_body,
        out_shape=jax.ShapeDtypeStruct((B, 1), jnp.float32),
    )(zp, xt, p['fc1_W'], p['fc1_b'], p['bnf1_g'], p['bnf1_b'],
      p['fc2_W'], p['fc2_b'], p['bnf2_g'], p['bnf2_b'],
      p['fc3_W'], p['fc3_b'], p['bnf3_g'], p['bnf3_b'],
      p['out_W'], p['out_b'])


def _bn_ncl(x, g, b):
    m = jnp.mean(x, (0, 2), keepdims=True)
    v = jnp.mean((x - m) ** 2, (0, 2), keepdims=True)
    return g[None, :, None] * (x - m) / jnp.sqrt(v + 1e-5) + b[None, :, None]


def _conv1d(x, W, b):
    y = jax.lax.conv_general_dilated(x, W, (1,), 'VALID',
                                     dimension_numbers=('NCH', 'OIH', 'NCH'))
    return y + b[None, :, None]


def _maxpool3(x):
    return jax.lax.reduce_window(x, -jnp.inf, jax.lax.max, (1, 1, 3), (1, 1, 3), 'VALID')


def kernel(x, edge_index, batch, target, params, eps):
    p = params
    src2 = jnp.concatenate([edge_index[0].astype(jnp.int32),
                            jnp.arange(N, dtype=jnp.int32)])
    dst2 = jnp.concatenate([edge_index[1].astype(jnp.int32),
                            jnp.arange(N, dtype=jnp.int32)])
    dst_s, src_s = jax.lax.sort((dst2, src2), num_keys=1)

    row_start = jnp.searchsorted(dst_s, jnp.arange(N + 1, dtype=jnp.int32))
    deg = jnp.diff(row_start).astype(jnp.float32)
    dinv = jax.lax.rsqrt(deg)

    npad = EPAD - (E + N)
    src_pad = jnp.concatenate([src_s, jnp.full((npad,), ZROW, jnp.int32)])
    dst_pad = jnp.concatenate([dst_s, jnp.full((npad,), jnp.int32(2**30), jnp.int32)])

    qb = jnp.searchsorted(dst_s, jnp.arange(0, NPAD + 1, QROWS, dtype=jnp.int32))
    qb = qb.at[NQ].set(EPAD)
    sblk = qb[:NQ] // KB
    nblk = (qb[1:] - sblk * KB + KB - 1) // KB
    tab = jnp.zeros((NQ, 16), jnp.int32)
    tab = tab.at[:, 0].set(sblk).at[:, 1].set(nblk).reshape(16 * NQ)

    dinv_c = dinv[:, None]

    def gcn_layer(h, W, b):
        u = dinv_c * jnp.dot(h, W, precision=_PREC)
        u = jnp.concatenate([u, jnp.zeros((UROWS - N, H), jnp.float32)])
        sfull = _propagate(u, src_pad, dst_pad, tab)
        return dinv_c * sfull[:N] + b

    identity = x @ p['res_W'] + p['res_b']
    h = jax.nn.relu(_bn_rows(gcn_layer(x, p['conv1_W'], p['conv1_b']), p['bn1_g'], p['bn1_b']))
    h = jax.nn.relu(_bn_rows(gcn_layer(h, p['conv2_W'], p['conv2_b']), p['bn2_g'], p['bn2_b']))
    h = jax.nn.relu(_bn_rows(gcn_layer(h, p['conv3_W'], p['conv3_b']), p['bn3_g'], p['bn3_b']))
    h = jax.nn.relu(_bn_rows(gcn_layer(h, p['conv4_W'], p['conv4_b']), p['bn4_g'], p['bn4_b']) + identity)
    mu = h @ p['mu_W'] + p['mu_b']
    logvar = h @ p['lv_W'] + p['lv_b']
    z = mu + eps * jnp.exp(0.5 * logvar)
    zp = jax.ops.segment_sum(z, batch, num_segments=B)
    m = jnp.mean(zp, -1, keepdims=True)
    v = jnp.mean((zp - m) ** 2, -1, keepdims=True)
    zp = p['ln_g'] * (zp - m) / jnp.sqrt(v + 1e-5) + p['ln_b']

    t = target[:, None, :]
    c = _maxpool3(jax.nn.relu(_bn_ncl(_conv1d(t, p['cxt1_W'], p['cxt1_b']), p['bnxt1_g'], p['bnxt1_b'])))
    c = _maxpool3(jax.nn.relu(_bn_ncl(_conv1d(c, p['cxt2_W'], p['cxt2_b']), p['bnxt2_g'], p['bnxt2_b'])))
    c = _maxpool3(jax.nn.relu(_bn_ncl(_conv1d(c, p['cxt3_W'], p['cxt3_b']), p['bnxt3_g'], p['bnxt3_b'])))
    xt = c.reshape(c.shape[0], -1) @ p['fc1xt_W'] + p['fc1xt_b']

    out = _head(zp, xt, p)
    return (out, zp)


# speedup vs baseline: 3.8460x; 1.0066x over previous
import functools

import jax
import jax.numpy as jnp
from jax import lax
from jax.experimental import pallas as pl
from jax.experimental.pallas import tpu as pltpu
from jax.experimental.pallas import tpu_sc as plsc

N = 50000
E = 800000
B = 256
L = 730
D_IN = 78
H = 128

_PREC = jax.lax.Precision.HIGHEST

QROWS = 8448
NQ = 6
NPAD = NQ * QROWS
UROWS = 50048
ZROW = 50000
TRASH = QROWS
BUFROWS = QROWS + 16
KB = 128
EPAD = 850048
NBLK = EPAD // KB
TROWS = QROWS // 16
ZROWS = 64
ZCHUNKS = [(o, min(ZROWS, TROWS - o)) for o in range(0, TROWS, ZROWS)]


def _propagate_body(u_hbm, srcs_hbm, dsts_hbm, tab_hbm, out_hbm,
                    tab_v, src_v, dst_v, idxl_v, rows_v, zero_v, buf_sh, sem):
    c = lax.axis_index("c")
    s = lax.axis_index("s")
    pltpu.sync_copy(tab_hbm, tab_v)
    tabs = [tab_v[pl.ds(16 * q, 16)] for q in range(NQ)]
    zvec = jnp.zeros((16,), jnp.float32)

    @pl.loop(0, ZROWS)
    def _(r):
        @pl.loop(0, H, step=16)
        def _(f):
            zero_v[r, pl.ds(f, 16)] = zvec

    for qi in range(NQ // 2):
        q = (NQ // 2) * c + qi
        qbase = q * QROWS
        tq = jnp.where(c == 0, tabs[qi], tabs[NQ // 2 + qi])
        sblk = tq[0]
        nblk = tq[1]

        for zo, zn in ZCHUNKS:
            zoff = pl.multiple_of(s * TROWS + zo, 8)
            pltpu.sync_copy(zero_v.at[pl.ds(0, zn)],
                            buf_sh.at[pl.ds(zoff, zn)])

        plsc.subcore_barrier()

        nmine = (nblk - s + 15) // 16

        @pl.loop(0, nmine)
        def _(i):
            blk = sblk + s + i * 16
            off = pl.multiple_of(blk * KB, KB)
            pltpu.sync_copy(srcs_hbm.at[pl.ds(off, KB)], src_v)
            pltpu.sync_copy(dsts_hbm.at[pl.ds(off, KB)], dst_v)
            for j in range(KB // 16):
                d = dst_v[pl.ds(j * 16, 16)]
                in_q = (d >= qbase) & (d < qbase + QROWS)
                loc = jnp.where(in_q, d - qbase, TRASH)
                idxl_v[pl.ds(j * 16, 16)] = loc
            pltpu.async_copy(u_hbm.at[src_v], rows_v, sem).wait()
            pltpu.sync_copy(rows_v, buf_sh.at[idxl_v], add=True)

        plsc.subcore_barrier()

        pltpu.sync_copy(buf_sh.at[pl.ds(pl.multiple_of(s * TROWS, 8), TROWS)],
                        out_hbm.at[pl.ds(pl.multiple_of(qbase + s * TROWS, 8), TROWS)])


def _propagate(u, srcs, dsts, tab):
    mesh = plsc.VectorSubcoreMesh(core_axis_name="c", subcore_axis_name="s")
    kern = pl.kernel(
        _propagate_body,
        out_type=jax.ShapeDtypeStruct((NPAD, H), jnp.float32),
        mesh=mesh,
        scratch_types=[
            pltpu.VMEM((16 * NQ,), jnp.int32),
            pltpu.VMEM((KB,), jnp.int32),
            pltpu.VMEM((KB,), jnp.int32),
            pltpu.VMEM((KB,), jnp.int32),
            pltpu.VMEM((KB, H), jnp.float32),
            pltpu.VMEM((ZROWS, H), jnp.float32),
            pltpu.VMEM_SHARED((BUFROWS, H), jnp.float32),
            pltpu.SemaphoreType.DMA,
        ],
    )
    return kern(u, srcs, dsts, tab)


def _bn_rows(x, g, b):
    m = jnp.mean(x, 0)
    v = jnp.mean((x - m) ** 2, 0)
    return g * (x - m) / jnp.sqrt(v + 1e-5) + b


def _head_body(zp_ref, xt_ref, w1_ref, b1_ref, g1_ref, bb1_ref,
               w2_ref, b2_ref, g2_ref, bb2_ref,
               w3_ref, b3_ref, g3_ref, bb3_ref,
               wo_ref, bo_ref, out_ref):
    xc = jnp.concatenate([zp_ref[...], xt_ref[...]], axis=1)
    h1 = jnp.dot(xc, w1_ref[...], precision=_PREC) + b1_ref[...]
    h1 = jax.nn.relu(_bn_rows(h1, g1_ref[...], bb1_ref[...]))
    h2 = jnp.dot(h1, w2_ref[...], precision=_PREC) + b2_ref[...]
    h2 = jax.nn.relu(_bn_rows(h2, g2_ref[...], bb2_ref[...]))
    h3 = jnp.dot(h2, w3_ref[...], precision=_PREC) + b3_ref[...]
    h3 = jax.nn.relu(_bn_rows(h3, g3_ref[...], bb3_ref[...]))
    out_ref[...] = jnp.dot(h3, wo_ref[...], precision=_PREC) + bo_ref[...]


def _head(zp, xt, p):
    return pl.pallas_call(
        _head_body,
        out_shape=jax.ShapeDtypeStruct((B, 1), jnp.float32),
    )(zp, xt, p['fc1_W'], p['fc1_b'], p['bnf1_g'], p['bnf1_b'],
      p['fc2_W'], p['fc2_b'], p['bnf2_g'], p['bnf2_b'],
      p['fc3_W'], p['fc3_b'], p['bnf3_g'], p['bnf3_b'],
      p['out_W'], p['out_b'])


def _bn_ncl(x, g, b):
    m = jnp.mean(x, (0, 2), keepdims=True)
    v = jnp.mean((x - m) ** 2, (0, 2), keepdims=True)
    return g[None, :, None] * (x - m) / jnp.sqrt(v + 1e-5) + b[None, :, None]


def _conv1d(x, W, b):
    y = jax.lax.conv_general_dilated(x, W, (1,), 'VALID',
                                     dimension_numbers=('NCH', 'OIH', 'NCH'))
    return y + b[None, :, None]


def _maxpool3(x):
    return jax.lax.reduce_window(x, -jnp.inf, jax.lax.max, (1, 1, 3), (1, 1, 3), 'VALID')


def kernel(x, edge_index, batch, target, params, eps):
    p = params
    src2 = jnp.concatenate([edge_index[0].astype(jnp.int32),
                            jnp.arange(N, dtype=jnp.int32)])
    dst2 = jnp.concatenate([edge_index[1].astype(jnp.int32),
                            jnp.arange(N, dtype=jnp.int32)])
    key = (dst2.astype(jnp.uint32) << jnp.uint32(16)) | src2.astype(jnp.uint32)
    key_s = jax.lax.sort(key)
    dst_s = (key_s >> jnp.uint32(16)).astype(jnp.int32)
    src_s = (key_s & jnp.uint32(0xFFFF)).astype(jnp.int32)

    row_start = jnp.searchsorted(dst_s, jnp.arange(N + 1, dtype=jnp.int32))
    deg = jnp.diff(row_start).astype(jnp.float32)
    dinv = jax.lax.rsqrt(deg)

    npad = EPAD - (E + N)
    src_pad = jnp.concatenate([src_s, jnp.full((npad,), ZROW, jnp.int32)])
    dst_pad = jnp.concatenate([dst_s, jnp.full((npad,), jnp.int32(2**30), jnp.int32)])

    qb = jnp.searchsorted(dst_s, jnp.arange(0, NPAD + 1, QROWS, dtype=jnp.int32))
    qb = qb.at[NQ].set(EPAD)
    sblk = qb[:NQ] // KB
    nblk = (qb[1:] - sblk * KB + KB - 1) // KB
    tab = jnp.zeros((NQ, 16), jnp.int32)
    tab = tab.at[:, 0].set(sblk).at[:, 1].set(nblk).reshape(16 * NQ)

    dinv_c = dinv[:, None]

    def gcn_layer(h, W, b):
        u = dinv_c * jnp.dot(h, W, precision=_PREC)
        u = jnp.concatenate([u, jnp.zeros((UROWS - N, H), jnp.float32)])
        sfull = _propagate(u, src_pad, dst_pad, tab)
        return dinv_c * sfull[:N] + b

    identity = x @ p['res_W'] + p['res_b']
    h = jax.nn.relu(_bn_rows(gcn_layer(x, p['conv1_W'], p['conv1_b']), p['bn1_g'], p['bn1_b']))
    h = jax.nn.relu(_bn_rows(gcn_layer(h, p['conv2_W'], p['conv2_b']), p['bn2_g'], p['bn2_b']))
    h = jax.nn.relu(_bn_rows(gcn_layer(h, p['conv3_W'], p['conv3_b']), p['bn3_g'], p['bn3_b']))
    h = jax.nn.relu(_bn_rows(gcn_layer(h, p['conv4_W'], p['conv4_b']), p['bn4_g'], p['bn4_b']) + identity)
    mu = h @ p['mu_W'] + p['mu_b']
    logvar = h @ p['lv_W'] + p['lv_b']
    z = mu + eps * jnp.exp(0.5 * logvar)
    zp = jax.ops.segment_sum(z, batch, num_segments=B)
    m = jnp.mean(zp, -1, keepdims=True)
    v = jnp.mean((zp - m) ** 2, -1, keepdims=True)
    zp = p['ln_g'] * (zp - m) / jnp.sqrt(v + 1e-5) + p['ln_b']

    t = target[:, None, :]
    c = _maxpool3(jax.nn.relu(_bn_ncl(_conv1d(t, p['cxt1_W'], p['cxt1_b']), p['bnxt1_g'], p['bnxt1_b'])))
    c = _maxpool3(jax.nn.relu(_bn_ncl(_conv1d(c, p['cxt2_W'], p['cxt2_b']), p['bnxt2_g'], p['bnxt2_b'])))
    c = _maxpool3(jax.nn.relu(_bn_ncl(_conv1d(c, p['cxt3_W'], p['cxt3_b']), p['bnxt3_g'], p['bnxt3_b'])))
    xt = c.reshape(c.shape[0], -1) @ p['fc1xt_W'] + p['fc1xt_b']

    out = _head(zp, xt, p)
    return (out, zp)
